# baseline (speedup 1.0000x reference)
import jax
import jax.numpy as jnp
from jax.experimental import pallas as pl
from jax.experimental.pallas import tpu as pltpu


def kernel(sequence, positions):
    B, S, D = sequence.shape
    _, N = positions.shape
    pos = positions.astype(jnp.int32)

    def body(idx_ref, seq_ref, out_ref, sem):
        copies = []
        for b in range(B):
            for r in range(N):
                idx = idx_ref[b, r]
                cp = pltpu.make_async_copy(
                    seq_ref.at[b, pl.ds(idx, 1)],
                    out_ref.at[b, pl.ds(r, 1)],
                    sem,
                )
                cp.start()
                copies.append(cp)
        for cp in copies:
            cp.wait()

    return pl.pallas_call(
        body,
        grid_spec=pltpu.PrefetchScalarGridSpec(
            num_scalar_prefetch=1,
            grid=(1,),
            in_specs=[pl.BlockSpec(memory_space=pl.ANY)],
            out_specs=pl.BlockSpec(memory_space=pl.ANY),
            scratch_shapes=[pltpu.SemaphoreType.DMA],
        ),
        out_shape=jax.ShapeDtypeStruct((B, N, D), jnp.float32),
    )(pos, sequence)


# speedup vs baseline: 8.7319x; 8.7319x over previous
import jax
import jax.numpy as jnp
from jax.experimental import pallas as pl
from jax.experimental.pallas import tpu as pltpu


def kernel(sequence, positions):
    B, S, D = sequence.shape
    _, N = positions.shape
    pos = positions.astype(jnp.int32)

    def body(idx_ref, seq_ref, out_ref, sem):
        b = pl.program_id(0)
        copies = []
        for r in range(N):
            idx = idx_ref[b, r]
            cp = pltpu.make_async_copy(
                seq_ref.at[b, pl.ds(idx, 1)],
                out_ref.at[0, pl.ds(r, 1)],
                sem,
            )
            cp.start()
            copies.append(cp)
        for cp in copies:
            cp.wait()

    return pl.pallas_call(
        body,
        grid_spec=pltpu.PrefetchScalarGridSpec(
            num_scalar_prefetch=1,
            grid=(B,),
            in_specs=[pl.BlockSpec(memory_space=pl.ANY)],
            out_specs=pl.BlockSpec((1, N, D), lambda b, idx_ref: (b, 0, 0)),
            scratch_shapes=[pltpu.SemaphoreType.DMA],
        ),
        out_shape=jax.ShapeDtypeStruct((B, N, D), jnp.float32),
        compiler_params=pltpu.CompilerParams(
            dimension_semantics=("parallel",),
        ),
    )(pos, sequence)
